# baseline (speedup 1.0000x reference)
import functools

import jax
import jax.numpy as jnp
from jax import lax
from jax.experimental import pallas as pl
from jax.experimental.pallas import tpu as pltpu
from jax.experimental.pallas import tpu_sc as plsc

E = 16
TOPK = 2
NGROUP = 4
GSIZE = E // NGROUP
TOPK_GROUP = 2
D = 2048
FF = 1024
SCALE = 2.5
T = 2048
P = T * TOPK
TILE = 128
PROWS = P + E * TILE
NTILES = PROWS // TILE
TB = 256
NSC = 32

_NEG = -1e30


def _router_body(x_ref, rw_ref, bias_ref, i1_ref, i2_ref, w1_ref, w2_ref):
    xb = x_ref[...]
    rw = rw_ref[...]
    logits = lax.dot_general(xb, rw, (((1,), (1,)), ((), ())),
                             preferred_element_type=jnp.float32)
    scores = jax.nn.sigmoid(logits)
    s4c = scores + bias_ref[...]
    lanes = lax.broadcasted_iota(jnp.int32, (TB, E), 1)

    def top1(vals):
        m = jnp.max(vals, axis=1, keepdims=True)
        i = jnp.min(jnp.where(vals == m, lanes, E), axis=1, keepdims=True)
        return m, i

    gs = jnp.where(lanes < NGROUP, 0.0, _NEG)
    for g in range(NGROUP):
        vals = jnp.where(lanes // GSIZE == g, s4c, _NEG)
        m1, i1 = top1(vals)
        m2, _ = top1(jnp.where(lanes == i1, _NEG, vals))
        gs = jnp.where(lanes == g, m1 + m2, gs)
    _, gi1 = top1(gs)
    _, gi2 = top1(jnp.where(lanes == gi1, _NEG, gs))
    sel = (lanes // GSIZE == gi1) | (lanes // GSIZE == gi2)
    masked = jnp.where(sel, s4c, 0.0)
    m1, i1 = top1(masked)
    _, i2 = top1(jnp.where(lanes == i1, _NEG, masked))
    w1 = jnp.sum(jnp.where(lanes == i1, scores, 0.0), axis=1, keepdims=True)
    w2 = jnp.sum(jnp.where(lanes == i2, scores, 0.0), axis=1, keepdims=True)
    norm = SCALE / (w1 + w2 + 1e-20)
    i1_ref[...] = i1
    i2_ref[...] = i2
    w1_ref[...] = w1 * norm
    w2_ref[...] = w2 * norm


def _run_router(x2, router_w, bias2):
    outs = jax.ShapeDtypeStruct((T, 1), jnp.int32), jax.ShapeDtypeStruct((T, 1), jnp.int32), \
        jax.ShapeDtypeStruct((T, 1), jnp.float32), jax.ShapeDtypeStruct((T, 1), jnp.float32)
    col = pl.BlockSpec((TB, 1), lambda i: (i, 0))
    return pl.pallas_call(
        _router_body,
        grid=(T // TB,),
        in_specs=[pl.BlockSpec((TB, D), lambda i: (i, 0)),
                  pl.BlockSpec((E, D), lambda i: (0, 0)),
                  pl.BlockSpec((1, E), lambda i: (0, 0))],
        out_specs=(col, col, col, col),
        out_shape=outs,
    )(x2, router_w, bias2)


def _plan_body(i1_ref, i2_ref, pos0_ref, pos1_ref, te_ref):
    lanes = lax.broadcasted_iota(jnp.int32, (T, E), 1)
    ohA = (i1_ref[...] == lanes).astype(jnp.int32)
    ohB = (i2_ref[...] == lanes).astype(jnp.int32)
    S = ohA + ohB
    counts = jnp.sum(S, axis=0, keepdims=True)
    padded = ((counts + (TILE - 1)) >> 7) << 7
    r16 = lax.broadcasted_iota(jnp.int32, (E, E), 0)
    c16 = lax.broadcasted_iota(jnp.int32, (E, E), 1)
    lt = (r16 <= c16).astype(jnp.float32)
    seg_incl = lax.dot_general(padded.astype(jnp.float32), lt,
                               (((1,), (0,)), ((), ())),
                               preferred_element_type=jnp.float32,
                               precision=lax.Precision.HIGHEST).astype(jnp.int32)
    seg_start = seg_incl - padded
    csum = S
    sh = 1
    while sh < T:
        shifted = jnp.concatenate(
            [jnp.zeros((sh, E), jnp.int32), csum[: T - sh, :]], axis=0)
        csum = csum + shifted
        sh *= 2
    excl = csum - S
    rank0 = jnp.sum(ohA * excl, axis=1, keepdims=True)
    rank1 = jnp.sum(ohB * excl, axis=1, keepdims=True)
    seg0 = jnp.sum(ohA * seg_start, axis=1, keepdims=True)
    seg1 = jnp.sum(ohB * seg_start, axis=1, keepdims=True)
    pos0_ref[...] = seg0 + rank0
    pos1_ref[...] = seg1 + rank1
    ts = lax.broadcasted_iota(jnp.int32, (NTILES, E), 0) * TILE
    te = jnp.sum((ts >= seg_incl).astype(jnp.int32), axis=1, keepdims=True)
    te_ref[...] = jnp.minimum(te, E - 1)


def _run_plan(i1, i2):
    col = pl.BlockSpec((T, 1), lambda: (0, 0))
    return pl.pallas_call(
        _plan_body,
        in_specs=[col, col],
        out_specs=(col, col, pl.BlockSpec((NTILES, 1), lambda: (0, 0))),
        out_shape=(jax.ShapeDtypeStruct((T, 1), jnp.int32),
                   jax.ShapeDtypeStruct((T, 1), jnp.int32),
                   jax.ShapeDtypeStruct((NTILES, 1), jnp.int32)),
    )(i1, i2)


def _invert_body(p0r_ref, p1r_ref, g1_ref, g2_ref, rt_ref, rg_ref):
    i = pl.program_id(0)
    rowid = i * TILE + lax.broadcasted_iota(jnp.int32, (TILE, T), 0)
    oh0 = (p0r_ref[...] == rowid).astype(jnp.float32)
    oh1 = (p1r_ref[...] == rowid).astype(jnp.float32)
    tokf = lax.broadcasted_iota(jnp.int32, (T, 1), 0).astype(jnp.float32)
    dn = (((1,), (0,)), ((), ()))
    tok = lax.dot_general(oh0 + oh1, tokf, dn,
                          preferred_element_type=jnp.float32,
                          precision=lax.Precision.HIGHEST)
    gate = lax.dot_general(oh0, g1_ref[...], dn,
                           preferred_element_type=jnp.float32,
                           precision=lax.Precision.HIGHEST) + \
        lax.dot_general(oh1, g2_ref[...], dn,
                        preferred_element_type=jnp.float32,
                        precision=lax.Precision.HIGHEST)
    rt_ref[...] = tok.astype(jnp.int32)
    rg_ref[...] = gate


def _run_invert(pos0, pos1, w1, w2):
    row = pl.BlockSpec((1, T), lambda i: (0, 0))
    col = pl.BlockSpec((T, 1), lambda i: (0, 0))
    blk = pl.BlockSpec((TILE, 1), lambda i: (i, 0))
    return pl.pallas_call(
        _invert_body,
        grid=(NTILES,),
        in_specs=[row, row, col, col],
        out_specs=(blk, blk),
        out_shape=(jax.ShapeDtypeStruct((PROWS, 1), jnp.int32),
                   jax.ShapeDtypeStruct((PROWS, 1), jnp.float32)),
    )(pos0.reshape(1, T), pos1.reshape(1, T), w1, w2)


def _gather_body(x_hbm, rt_hbm, xs_hbm, raw_v, idx_v, rows_v, sem):
    wid = lax.axis_index("s") * 2 + lax.axis_index("c")
    per = PROWS // NSC
    chunk = 48
    for ci in range(per // chunk):
        o = wid * per + ci * chunk
        pltpu.sync_copy(rt_hbm.at[pl.ds(o, chunk)], raw_v)
        for c in range(chunk // 16):
            idx_v[pl.ds(c * 16, 16)] = raw_v[pl.ds(c * 16, 16)] & (T - 1)
        cp = pltpu.make_async_copy(x_hbm.at[idx_v], rows_v, sem)
        cp.start()
        cp.wait()
        pltpu.sync_copy(rows_v, xs_hbm.at[pl.ds(o, chunk)])


def _run_gather(x2, row_token):
    mesh = plsc.VectorSubcoreMesh(core_axis_name="c", subcore_axis_name="s")
    f = pl.kernel(
        _gather_body,
        out_type=jax.ShapeDtypeStruct((PROWS, D), jnp.float32),
        mesh=mesh,
        scratch_types=[pltpu.VMEM((48,), jnp.int32),
                       pltpu.VMEM((48,), jnp.int32),
                       pltpu.VMEM((48, D), jnp.float32),
                       pltpu.SemaphoreType.DMA],
    )
    return f(x2, row_token)


def _ffn_body(te_ref, xs_ref, gate_ref, wg_ref, wu_ref, wd_ref, out_ref):
    xb = xs_ref[...]
    g = lax.dot_general(xb, wg_ref[0], (((1,), (1,)), ((), ())),
                        preferred_element_type=jnp.float32,
                        precision=lax.Precision.DEFAULT)
    u = lax.dot_general(xb, wu_ref[0], (((1,), (1,)), ((), ())),
                        preferred_element_type=jnp.float32,
                        precision=lax.Precision.DEFAULT)
    h = g * jax.nn.sigmoid(g) * u
    o = lax.dot_general(h, wd_ref[0], (((1,), (1,)), ((), ())),
                        preferred_element_type=jnp.float32,
                        precision=lax.Precision.DEFAULT)
    out_ref[...] = o * gate_ref[...]


def _run_ffn(te, xs, row_gate_col, Wg, Wu, Wd):
    spec = pltpu.PrefetchScalarGridSpec(
        num_scalar_prefetch=1,
        grid=(NTILES,),
        in_specs=[
            pl.BlockSpec((TILE, D), lambda i, te: (i, 0)),
            pl.BlockSpec((TILE, 1), lambda i, te: (i, 0)),
            pl.BlockSpec((1, FF, D), lambda i, te: (te[i], 0, 0)),
            pl.BlockSpec((1, FF, D), lambda i, te: (te[i], 0, 0)),
            pl.BlockSpec((1, D, FF), lambda i, te: (te[i], 0, 0)),
        ],
        out_specs=pl.BlockSpec((TILE, D), lambda i, te: (i, 0)),
    )
    return pl.pallas_call(
        _ffn_body,
        grid_spec=spec,
        out_shape=jax.ShapeDtypeStruct((PROWS, D), jnp.float32),
    )(te, xs, row_gate_col, Wg, Wu, Wd)


def _combine_body(h_hbm, p0_hbm, p1_hbm, g_hbm, p0_v, p1_v, a_v, b_v, semA,
                  semB):
    wid = lax.axis_index("s") * 2 + lax.axis_index("c")
    per = T // NSC
    chunk = 16
    for ci in range(per // chunk):
        o = wid * per + ci * chunk
        pltpu.sync_copy(p0_hbm.at[pl.ds(o, chunk)], p0_v)
        pltpu.sync_copy(p1_hbm.at[pl.ds(o, chunk)], p1_v)
        cpA = pltpu.make_async_copy(h_hbm.at[p0_v], a_v, semA)
        cpB = pltpu.make_async_copy(h_hbm.at[p1_v], b_v, semB)
        cpA.start()
        cpB.start()
        cpA.wait()
        cpB.wait()

        def addb(i, carry):
            r = i >> 7
            c = i & 127
            plsc.addupdate(a_v.at[r, pl.ds(c * 16, 16)],
                           b_v[r, pl.ds(c * 16, 16)])
            return carry

        lax.fori_loop(0, chunk * (D // 16), addb, 0)
        pltpu.sync_copy(a_v, g_hbm.at[pl.ds(o, chunk)])


def _run_combine(h, pos0, pos1):
    mesh = plsc.VectorSubcoreMesh(core_axis_name="c", subcore_axis_name="s")
    f = pl.kernel(
        _combine_body,
        out_type=jax.ShapeDtypeStruct((T, D), jnp.float32),
        mesh=mesh,
        scratch_types=[pltpu.VMEM((16,), jnp.int32),
                       pltpu.VMEM((16,), jnp.int32),
                       pltpu.VMEM((16, D), jnp.float32),
                       pltpu.VMEM((16, D), jnp.float32),
                       pltpu.SemaphoreType.DMA,
                       pltpu.SemaphoreType.DMA],
    )
    return f(h, pos0, pos1)


def _shared_body(x_ref, gin_ref, sg_ref, su_ref, sd_ref, out_ref):
    xb = x_ref[...]
    g = lax.dot_general(xb, sg_ref[...], (((1,), (1,)), ((), ())),
                        preferred_element_type=jnp.float32,
                        precision=lax.Precision.DEFAULT)
    u = lax.dot_general(xb, su_ref[...], (((1,), (1,)), ((), ())),
                        preferred_element_type=jnp.float32,
                        precision=lax.Precision.DEFAULT)
    h = g * jax.nn.sigmoid(g) * u
    o = lax.dot_general(h, sd_ref[...], (((1,), (1,)), ((), ())),
                        preferred_element_type=jnp.float32,
                        precision=lax.Precision.DEFAULT)
    out_ref[...] = o + gin_ref[...]


def _run_shared(x2, gsum, Sg, Su, Sd):
    return pl.pallas_call(
        _shared_body,
        grid=(T // TILE,),
        in_specs=[pl.BlockSpec((TILE, D), lambda i: (i, 0)),
                  pl.BlockSpec((TILE, D), lambda i: (i, 0)),
                  pl.BlockSpec((FF, D), lambda i: (0, 0)),
                  pl.BlockSpec((FF, D), lambda i: (0, 0)),
                  pl.BlockSpec((D, FF), lambda i: (0, 0))],
        out_specs=pl.BlockSpec((TILE, D), lambda i: (i, 0)),
        out_shape=jax.ShapeDtypeStruct((T, D), jnp.float32),
    )(x2, gsum, Sg, Su, Sd)


def kernel(hidden_states, router_w, bias, Wg, Wu, Wd, Sg, Su, Sd):
    orig_shape = hidden_states.shape
    x2 = hidden_states.reshape(T, D)
    i1, i2, w1, w2 = _run_router(x2, router_w, bias.reshape(1, E))
    pos0, pos1, te_col = _run_plan(i1, i2)
    row_token, row_gate = _run_invert(pos0, pos1, w1, w2)
    xs = _run_gather(x2, row_token.reshape(PROWS))
    h = _run_ffn(te_col.reshape(NTILES), xs, row_gate, Wg, Wu, Wd)
    gsum = _run_combine(h, pos0.reshape(T), pos1.reshape(T))
    out = _run_shared(x2, gsum, Sg, Su, Sd)
    return out.reshape(orig_shape)


# speedup vs baseline: 1.4081x; 1.4081x over previous
import functools

import jax
import jax.numpy as jnp
from jax import lax
from jax.experimental import pallas as pl
from jax.experimental.pallas import tpu as pltpu
from jax.experimental.pallas import tpu_sc as plsc

E = 16
TOPK = 2
NGROUP = 4
GSIZE = E // NGROUP
TOPK_GROUP = 2
D = 2048
FF = 1024
SCALE = 2.5
T = 2048
P = T * TOPK
TILE = 128
PROWS = P + E * TILE
NTILES = PROWS // TILE
TB = 256
NSC = 32

_NEG = -1e30


def _router_body(x_ref, rw_ref, bias_ref, i1_ref, i2_ref, w1_ref, w2_ref):
    xb = x_ref[...]
    rw = rw_ref[...]
    logits = lax.dot_general(xb, rw, (((1,), (1,)), ((), ())),
                             preferred_element_type=jnp.float32)
    scores = jax.nn.sigmoid(logits)
    s4c = scores + bias_ref[...]
    lanes = lax.broadcasted_iota(jnp.int32, (TB, E), 1)

    def top1(vals):
        m = jnp.max(vals, axis=1, keepdims=True)
        i = jnp.min(jnp.where(vals == m, lanes, E), axis=1, keepdims=True)
        return m, i

    gs = jnp.where(lanes < NGROUP, 0.0, _NEG)
    for g in range(NGROUP):
        vals = jnp.where(lanes // GSIZE == g, s4c, _NEG)
        m1, i1 = top1(vals)
        m2, _ = top1(jnp.where(lanes == i1, _NEG, vals))
        gs = jnp.where(lanes == g, m1 + m2, gs)
    _, gi1 = top1(gs)
    _, gi2 = top1(jnp.where(lanes == gi1, _NEG, gs))
    sel = (lanes // GSIZE == gi1) | (lanes // GSIZE == gi2)
    masked = jnp.where(sel, s4c, 0.0)
    m1, i1 = top1(masked)
    _, i2 = top1(jnp.where(lanes == i1, _NEG, masked))
    w1 = jnp.sum(jnp.where(lanes == i1, scores, 0.0), axis=1, keepdims=True)
    w2 = jnp.sum(jnp.where(lanes == i2, scores, 0.0), axis=1, keepdims=True)
    norm = SCALE / (w1 + w2 + 1e-20)
    i1_ref[...] = i1
    i2_ref[...] = i2
    w1_ref[...] = w1 * norm
    w2_ref[...] = w2 * norm


def _run_router(x2, router_w, bias2):
    outs = jax.ShapeDtypeStruct((T, 1), jnp.int32), jax.ShapeDtypeStruct((T, 1), jnp.int32), \
        jax.ShapeDtypeStruct((T, 1), jnp.float32), jax.ShapeDtypeStruct((T, 1), jnp.float32)
    col = pl.BlockSpec((TB, 1), lambda i: (i, 0))
    return pl.pallas_call(
        _router_body,
        grid=(T // TB,),
        in_specs=[pl.BlockSpec((TB, D), lambda i: (i, 0)),
                  pl.BlockSpec((E, D), lambda i: (0, 0)),
                  pl.BlockSpec((1, E), lambda i: (0, 0))],
        out_specs=(col, col, col, col),
        out_shape=outs,
    )(x2, router_w, bias2)


def _plan_body(i1_ref, i2_ref, pos0_ref, pos1_ref, te_ref):
    lanes = lax.broadcasted_iota(jnp.int32, (T, E), 1)
    ohA = (i1_ref[...] == lanes).astype(jnp.int32)
    ohB = (i2_ref[...] == lanes).astype(jnp.int32)
    S = ohA + ohB
    counts = jnp.sum(S, axis=0, keepdims=True)
    padded = ((counts + (TILE - 1)) >> 7) << 7
    r16 = lax.broadcasted_iota(jnp.int32, (E, E), 0)
    c16 = lax.broadcasted_iota(jnp.int32, (E, E), 1)
    lt = (r16 <= c16).astype(jnp.float32)
    seg_incl = lax.dot_general(padded.astype(jnp.float32), lt,
                               (((1,), (0,)), ((), ())),
                               preferred_element_type=jnp.float32,
                               precision=lax.Precision.HIGHEST).astype(jnp.int32)
    seg_start = seg_incl - padded
    csum = S
    sh = 1
    while sh < T:
        shifted = jnp.concatenate(
            [jnp.zeros((sh, E), jnp.int32), csum[: T - sh, :]], axis=0)
        csum = csum + shifted
        sh *= 2
    excl = csum - S
    rank0 = jnp.sum(ohA * excl, axis=1, keepdims=True)
    rank1 = jnp.sum(ohB * excl, axis=1, keepdims=True)
    seg0 = jnp.sum(ohA * seg_start, axis=1, keepdims=True)
    seg1 = jnp.sum(ohB * seg_start, axis=1, keepdims=True)
    pos0_ref[...] = seg0 + rank0
    pos1_ref[...] = seg1 + rank1
    ts = lax.broadcasted_iota(jnp.int32, (NTILES, E), 0) * TILE
    te = jnp.sum((ts >= seg_incl).astype(jnp.int32), axis=1, keepdims=True)
    te_ref[...] = jnp.minimum(te, E - 1)


def _run_plan(i1, i2):
    col = pl.BlockSpec((T, 1), lambda: (0, 0))
    return pl.pallas_call(
        _plan_body,
        in_specs=[col, col],
        out_specs=(col, col, pl.BlockSpec((NTILES, 1), lambda: (0, 0))),
        out_shape=(jax.ShapeDtypeStruct((T, 1), jnp.int32),
                   jax.ShapeDtypeStruct((T, 1), jnp.int32),
                   jax.ShapeDtypeStruct((NTILES, 1), jnp.int32)),
    )(i1, i2)


def _invert_body(p0r_ref, p1r_ref, g1r_ref, g2r_ref, rt_ref, rg_ref):
    i = pl.program_id(0)
    rowid = i * TILE + lax.broadcasted_iota(jnp.int32, (TILE, T), 0)
    oh0 = p0r_ref[...] == rowid
    oh1 = p1r_ref[...] == rowid
    ohe = oh0 | oh1
    lanesT = lax.broadcasted_iota(jnp.int32, (TILE, T), 1)
    tok = jnp.sum(jnp.where(ohe, lanesT, 0), axis=1, keepdims=True)
    matched = jnp.sum(ohe.astype(jnp.int32), axis=1, keepdims=True)
    rowcol = i * TILE + lax.broadcasted_iota(jnp.int32, (TILE, 1), 0)
    rt_ref[...] = jnp.where(matched == 0, rowcol & (T - 1), tok)
    gate = jnp.sum(jnp.where(oh0, g1r_ref[...], 0.0)
                   + jnp.where(oh1, g2r_ref[...], 0.0), axis=1, keepdims=True)
    rg_ref[...] = gate


def _run_invert(pos0, pos1, w1, w2):
    row = pl.BlockSpec((1, T), lambda i: (0, 0))
    blk = pl.BlockSpec((TILE, 1), lambda i: (i, 0))
    return pl.pallas_call(
        _invert_body,
        grid=(NTILES,),
        in_specs=[row, row, row, row],
        out_specs=(blk, blk),
        out_shape=(jax.ShapeDtypeStruct((PROWS, 1), jnp.int32),
                   jax.ShapeDtypeStruct((PROWS, 1), jnp.float32)),
    )(pos0.reshape(1, T), pos1.reshape(1, T),
      w1.reshape(1, T), w2.reshape(1, T))


def _gather_body(x_hbm, rt_hbm, xs_hbm, raw_v, idx_v, rows_v, sem):
    wid = lax.axis_index("s") * 2 + lax.axis_index("c")
    per = PROWS // NSC
    chunk = 48
    for ci in range(per // chunk):
        o = wid * per + ci * chunk
        pltpu.sync_copy(rt_hbm.at[pl.ds(o, chunk)], raw_v)
        for c in range(chunk // 16):
            idx_v[pl.ds(c * 16, 16)] = raw_v[pl.ds(c * 16, 16)] & (T - 1)
        cp = pltpu.make_async_copy(x_hbm.at[idx_v], rows_v, sem)
        cp.start()
        cp.wait()
        pltpu.sync_copy(rows_v, xs_hbm.at[pl.ds(o, chunk)])


def _run_gather(x2, row_token):
    mesh = plsc.VectorSubcoreMesh(core_axis_name="c", subcore_axis_name="s")
    f = pl.kernel(
        _gather_body,
        out_type=jax.ShapeDtypeStruct((PROWS, D), jnp.float32),
        mesh=mesh,
        scratch_types=[pltpu.VMEM((48,), jnp.int32),
                       pltpu.VMEM((48,), jnp.int32),
                       pltpu.VMEM((48, D), jnp.float32),
                       pltpu.SemaphoreType.DMA],
    )
    return f(x2, row_token)


def _ffn_body(te_ref, xs_ref, gate_ref, wg_ref, wu_ref, wd_ref, out_ref):
    xb = xs_ref[...]
    g = lax.dot_general(xb, wg_ref[0], (((1,), (1,)), ((), ())),
                        preferred_element_type=jnp.float32,
                        precision=lax.Precision.DEFAULT)
    u = lax.dot_general(xb, wu_ref[0], (((1,), (1,)), ((), ())),
                        preferred_element_type=jnp.float32,
                        precision=lax.Precision.DEFAULT)
    h = g * jax.nn.sigmoid(g) * u
    o = lax.dot_general(h, wd_ref[0], (((1,), (1,)), ((), ())),
                        preferred_element_type=jnp.float32,
                        precision=lax.Precision.DEFAULT)
    out_ref[...] = o * gate_ref[...]


def _run_ffn(te, xs, row_gate_col, Wg, Wu, Wd):
    spec = pltpu.PrefetchScalarGridSpec(
        num_scalar_prefetch=1,
        grid=(NTILES,),
        in_specs=[
            pl.BlockSpec((TILE, D), lambda i, te: (i, 0)),
            pl.BlockSpec((TILE, 1), lambda i, te: (i, 0)),
            pl.BlockSpec((1, FF, D), lambda i, te: (te[i], 0, 0)),
            pl.BlockSpec((1, FF, D), lambda i, te: (te[i], 0, 0)),
            pl.BlockSpec((1, D, FF), lambda i, te: (te[i], 0, 0)),
        ],
        out_specs=pl.BlockSpec((TILE, D), lambda i, te: (i, 0)),
    )
    return pl.pallas_call(
        _ffn_body,
        grid_spec=spec,
        out_shape=jax.ShapeDtypeStruct((PROWS, D), jnp.float32),
    )(te, xs, row_gate_col, Wg, Wu, Wd)


def _combine_body(h_hbm, p0_hbm, p1_hbm, g_hbm, p0_v, p1_v, a_v, b_v, semA,
                  semB):
    wid = lax.axis_index("s") * 2 + lax.axis_index("c")
    per = T // NSC
    chunk = 16
    for ci in range(per // chunk):
        o = wid * per + ci * chunk
        pltpu.sync_copy(p0_hbm.at[pl.ds(o, chunk)], p0_v)
        pltpu.sync_copy(p1_hbm.at[pl.ds(o, chunk)], p1_v)
        cpA = pltpu.make_async_copy(h_hbm.at[p0_v], a_v, semA)
        cpB = pltpu.make_async_copy(h_hbm.at[p1_v], b_v, semB)
        cpA.start()
        cpB.start()
        cpA.wait()
        cpB.wait()

        def addb(i, carry):
            r = i >> 7
            c = i & 127
            plsc.addupdate(a_v.at[r, pl.ds(c * 16, 16)],
                           b_v[r, pl.ds(c * 16, 16)])
            return carry

        lax.fori_loop(0, chunk * (D // 16), addb, 0)
        pltpu.sync_copy(a_v, g_hbm.at[pl.ds(o, chunk)])


def _run_combine(h, pos0, pos1):
    mesh = plsc.VectorSubcoreMesh(core_axis_name="c", subcore_axis_name="s")
    f = pl.kernel(
        _combine_body,
        out_type=jax.ShapeDtypeStruct((T, D), jnp.float32),
        mesh=mesh,
        scratch_types=[pltpu.VMEM((16,), jnp.int32),
                       pltpu.VMEM((16,), jnp.int32),
                       pltpu.VMEM((16, D), jnp.float32),
                       pltpu.VMEM((16, D), jnp.float32),
                       pltpu.SemaphoreType.DMA,
                       pltpu.SemaphoreType.DMA],
    )
    return f(h, pos0, pos1)


def _shared_body(x_ref, gin_ref, sg_ref, su_ref, sd_ref, out_ref):
    xb = x_ref[...]
    g = lax.dot_general(xb, sg_ref[...], (((1,), (1,)), ((), ())),
                        preferred_element_type=jnp.float32,
                        precision=lax.Precision.DEFAULT)
    u = lax.dot_general(xb, su_ref[...], (((1,), (1,)), ((), ())),
                        preferred_element_type=jnp.float32,
                        precision=lax.Precision.DEFAULT)
    h = g * jax.nn.sigmoid(g) * u
    o = lax.dot_general(h, sd_ref[...], (((1,), (1,)), ((), ())),
                        preferred_element_type=jnp.float32,
                        precision=lax.Precision.DEFAULT)
    out_ref[...] = o + gin_ref[...]


def _run_shared(x2, gsum, Sg, Su, Sd):
    return pl.pallas_call(
        _shared_body,
        grid=(T // TILE,),
        in_specs=[pl.BlockSpec((TILE, D), lambda i: (i, 0)),
                  pl.BlockSpec((TILE, D), lambda i: (i, 0)),
                  pl.BlockSpec((FF, D), lambda i: (0, 0)),
                  pl.BlockSpec((FF, D), lambda i: (0, 0)),
                  pl.BlockSpec((D, FF), lambda i: (0, 0))],
        out_specs=pl.BlockSpec((TILE, D), lambda i: (i, 0)),
        out_shape=jax.ShapeDtypeStruct((T, D), jnp.float32),
    )(x2, gsum, Sg, Su, Sd)


def kernel(hidden_states, router_w, bias, Wg, Wu, Wd, Sg, Su, Sd):
    orig_shape = hidden_states.shape
    x2 = hidden_states.reshape(T, D)
    i1, i2, w1, w2 = _run_router(x2, router_w, bias.reshape(1, E))
    pos0, pos1, te_col = _run_plan(i1, i2)
    row_token, row_gate = _run_invert(pos0, pos1, w1, w2)
    xs = _run_gather(x2, row_token.reshape(PROWS))
    h = _run_ffn(te_col.reshape(NTILES), xs, row_gate, Wg, Wu, Wd)
    gsum = _run_combine(h, pos0.reshape(T), pos1.reshape(T))
    out = _run_shared(x2, gsum, Sg, Su, Sd)
    return out.reshape(orig_shape)


# speedup vs baseline: 1.4254x; 1.0123x over previous
import functools

import jax
import jax.numpy as jnp
from jax import lax
from jax.experimental import pallas as pl
from jax.experimental.pallas import tpu as pltpu
from jax.experimental.pallas import tpu_sc as plsc

E = 16
TOPK = 2
NGROUP = 4
GSIZE = E // NGROUP
TOPK_GROUP = 2
D = 2048
FF = 1024
SCALE = 2.5
T = 2048
P = T * TOPK
TILE = 128
PROWS = P + E * TILE
NTILES = PROWS // TILE
TB = 256
NSC = 32

_NEG = -1e30


def _router_body(x_ref, rw_ref, bias_ref, i1_ref, i2_ref, w1_ref, w2_ref):
    xb = x_ref[...]
    rw = rw_ref[...]
    logits = lax.dot_general(xb, rw, (((1,), (1,)), ((), ())),
                             preferred_element_type=jnp.float32)
    scores = jax.nn.sigmoid(logits)
    s4c = scores + bias_ref[...]
    lanes = lax.broadcasted_iota(jnp.int32, (TB, E), 1)

    def top1(vals):
        m = jnp.max(vals, axis=1, keepdims=True)
        i = jnp.min(jnp.where(vals == m, lanes, E), axis=1, keepdims=True)
        return m, i

    gs = jnp.where(lanes < NGROUP, 0.0, _NEG)
    for g in range(NGROUP):
        vals = jnp.where(lanes // GSIZE == g, s4c, _NEG)
        m1, i1 = top1(vals)
        m2, _ = top1(jnp.where(lanes == i1, _NEG, vals))
        gs = jnp.where(lanes == g, m1 + m2, gs)
    _, gi1 = top1(gs)
    _, gi2 = top1(jnp.where(lanes == gi1, _NEG, gs))
    sel = (lanes // GSIZE == gi1) | (lanes // GSIZE == gi2)
    masked = jnp.where(sel, s4c, 0.0)
    m1, i1 = top1(masked)
    _, i2 = top1(jnp.where(lanes == i1, _NEG, masked))
    w1 = jnp.sum(jnp.where(lanes == i1, scores, 0.0), axis=1, keepdims=True)
    w2 = jnp.sum(jnp.where(lanes == i2, scores, 0.0), axis=1, keepdims=True)
    norm = SCALE / (w1 + w2 + 1e-20)
    i1_ref[...] = i1
    i2_ref[...] = i2
    w1_ref[...] = w1 * norm
    w2_ref[...] = w2 * norm


def _run_router(x2, router_w, bias2):
    outs = jax.ShapeDtypeStruct((T, 1), jnp.int32), jax.ShapeDtypeStruct((T, 1), jnp.int32), \
        jax.ShapeDtypeStruct((T, 1), jnp.float32), jax.ShapeDtypeStruct((T, 1), jnp.float32)
    col = pl.BlockSpec((TB, 1), lambda i: (i, 0))
    return pl.pallas_call(
        _router_body,
        grid=(T // TB,),
        in_specs=[pl.BlockSpec((TB, D), lambda i: (i, 0)),
                  pl.BlockSpec((E, D), lambda i: (0, 0)),
                  pl.BlockSpec((1, E), lambda i: (0, 0))],
        out_specs=(col, col, col, col),
        out_shape=outs,
    )(x2, router_w, bias2)


def _plan_body(i1_ref, i2_ref, pos0_ref, pos1_ref, te_ref):
    lanes = lax.broadcasted_iota(jnp.int32, (T, E), 1)
    ohA = (i1_ref[...] == lanes).astype(jnp.int32)
    ohB = (i2_ref[...] == lanes).astype(jnp.int32)
    S = ohA + ohB
    counts = jnp.sum(S, axis=0, keepdims=True)
    padded = ((counts + (TILE - 1)) >> 7) << 7
    r16 = lax.broadcasted_iota(jnp.int32, (E, E), 0)
    c16 = lax.broadcasted_iota(jnp.int32, (E, E), 1)
    lt = (r16 <= c16).astype(jnp.float32)
    seg_incl = lax.dot_general(padded.astype(jnp.float32), lt,
                               (((1,), (0,)), ((), ())),
                               preferred_element_type=jnp.float32,
                               precision=lax.Precision.HIGHEST).astype(jnp.int32)
    seg_start = seg_incl - padded
    csum = S
    sh = 1
    while sh < T:
        shifted = jnp.concatenate(
            [jnp.zeros((sh, E), jnp.int32), csum[: T - sh, :]], axis=0)
        csum = csum + shifted
        sh *= 2
    excl = csum - S
    rank0 = jnp.sum(ohA * excl, axis=1, keepdims=True)
    rank1 = jnp.sum(ohB * excl, axis=1, keepdims=True)
    seg0 = jnp.sum(ohA * seg_start, axis=1, keepdims=True)
    seg1 = jnp.sum(ohB * seg_start, axis=1, keepdims=True)
    pos0_ref[...] = seg0 + rank0
    pos1_ref[...] = seg1 + rank1
    ts = lax.broadcasted_iota(jnp.int32, (NTILES, E), 0) * TILE
    te = jnp.sum((ts >= seg_incl).astype(jnp.int32), axis=1, keepdims=True)
    te_ref[...] = jnp.minimum(te, E - 1)


def _run_plan(i1, i2):
    col = pl.BlockSpec((T, 1), lambda: (0, 0))
    return pl.pallas_call(
        _plan_body,
        in_specs=[col, col],
        out_specs=(col, col, pl.BlockSpec((NTILES, 1), lambda: (0, 0))),
        out_shape=(jax.ShapeDtypeStruct((T, 1), jnp.int32),
                   jax.ShapeDtypeStruct((T, 1), jnp.int32),
                   jax.ShapeDtypeStruct((NTILES, 1), jnp.int32)),
    )(i1, i2)


def _invert_body(p0r_ref, p1r_ref, g1r_ref, g2r_ref, rt_ref, rg_ref):
    i = pl.program_id(0)
    rowid = i * TILE + lax.broadcasted_iota(jnp.int32, (TILE, T), 0)
    oh0 = p0r_ref[...] == rowid
    oh1 = p1r_ref[...] == rowid
    ohe = oh0 | oh1
    lanesT = lax.broadcasted_iota(jnp.int32, (TILE, T), 1)
    tok = jnp.sum(jnp.where(ohe, lanesT, 0), axis=1, keepdims=True)
    matched = jnp.sum(ohe.astype(jnp.int32), axis=1, keepdims=True)
    rowcol = i * TILE + lax.broadcasted_iota(jnp.int32, (TILE, 1), 0)
    rt_ref[...] = jnp.where(matched == 0, rowcol & (T - 1), tok)
    gate = jnp.sum(jnp.where(oh0, g1r_ref[...], 0.0)
                   + jnp.where(oh1, g2r_ref[...], 0.0), axis=1, keepdims=True)
    rg_ref[...] = gate


def _run_invert(pos0, pos1, w1, w2):
    row = pl.BlockSpec((1, T), lambda i: (0, 0))
    blk = pl.BlockSpec((TILE, 1), lambda i: (i, 0))
    return pl.pallas_call(
        _invert_body,
        grid=(NTILES,),
        in_specs=[row, row, row, row],
        out_specs=(blk, blk),
        out_shape=(jax.ShapeDtypeStruct((PROWS, 1), jnp.int32),
                   jax.ShapeDtypeStruct((PROWS, 1), jnp.float32)),
    )(pos0.reshape(1, T), pos1.reshape(1, T),
      w1.reshape(1, T), w2.reshape(1, T))


def _gather_body(x_hbm, rt_hbm, xs_hbm, raw_v, idx_v, rows_v, sem):
    wid = lax.axis_index("s") * 2 + lax.axis_index("c")
    per = PROWS // NSC
    chunk = 48
    for ci in range(per // chunk):
        o = wid * per + ci * chunk
        pltpu.sync_copy(rt_hbm.at[pl.ds(o, chunk)], raw_v)
        for c in range(chunk // 16):
            idx_v[pl.ds(c * 16, 16)] = raw_v[pl.ds(c * 16, 16)] & (T - 1)
        cp = pltpu.make_async_copy(x_hbm.at[idx_v], rows_v, sem)
        cp.start()
        cp.wait()
        pltpu.sync_copy(rows_v, xs_hbm.at[pl.ds(o, chunk)])


def _run_gather(x2, row_token):
    mesh = plsc.VectorSubcoreMesh(core_axis_name="c", subcore_axis_name="s")
    f = pl.kernel(
        _gather_body,
        out_type=jax.ShapeDtypeStruct((PROWS, D), jnp.float32),
        mesh=mesh,
        scratch_types=[pltpu.VMEM((48,), jnp.int32),
                       pltpu.VMEM((48,), jnp.int32),
                       pltpu.VMEM((48, D), jnp.float32),
                       pltpu.SemaphoreType.DMA],
    )
    return f(x2, row_token)


def _ffn_body(te_ref, xs_ref, gate_ref, wg_ref, wu_ref, wd_ref, out_ref):
    xb = xs_ref[...]
    g = lax.dot_general(xb, wg_ref[0], (((1,), (1,)), ((), ())),
                        preferred_element_type=jnp.float32,
                        precision=lax.Precision.DEFAULT)
    u = lax.dot_general(xb, wu_ref[0], (((1,), (1,)), ((), ())),
                        preferred_element_type=jnp.float32,
                        precision=lax.Precision.DEFAULT)
    h = g * jax.nn.sigmoid(g) * u
    o = lax.dot_general(h, wd_ref[0], (((1,), (1,)), ((), ())),
                        preferred_element_type=jnp.float32,
                        precision=lax.Precision.DEFAULT)
    out_ref[...] = o * gate_ref[...]


def _run_ffn(te, xs, row_gate_col, Wg, Wu, Wd):
    spec = pltpu.PrefetchScalarGridSpec(
        num_scalar_prefetch=1,
        grid=(NTILES,),
        in_specs=[
            pl.BlockSpec((TILE, D), lambda i, te: (i, 0)),
            pl.BlockSpec((TILE, 1), lambda i, te: (i, 0)),
            pl.BlockSpec((1, FF, D), lambda i, te: (te[i], 0, 0)),
            pl.BlockSpec((1, FF, D), lambda i, te: (te[i], 0, 0)),
            pl.BlockSpec((1, D, FF), lambda i, te: (te[i], 0, 0)),
        ],
        out_specs=pl.BlockSpec((TILE, D), lambda i, te: (i, 0)),
    )
    return pl.pallas_call(
        _ffn_body,
        grid_spec=spec,
        out_shape=jax.ShapeDtypeStruct((PROWS, D), jnp.float32),
    )(te, xs, row_gate_col, Wg, Wu, Wd)


def _combine_body(h_hbm, s_hbm, p0_hbm, p1_hbm, out_hbm, p0_v, p1_v, a_v, b_v,
                  c_v, semA, semB, semC):
    wid = lax.axis_index("s") * 2 + lax.axis_index("c")
    per = T // NSC
    chunk = 16
    for ci in range(per // chunk):
        o = wid * per + ci * chunk
        pltpu.sync_copy(p0_hbm.at[pl.ds(o, chunk)], p0_v)
        pltpu.sync_copy(p1_hbm.at[pl.ds(o, chunk)], p1_v)
        cpA = pltpu.make_async_copy(h_hbm.at[p0_v], a_v, semA)
        cpB = pltpu.make_async_copy(h_hbm.at[p1_v], b_v, semB)
        cpC = pltpu.make_async_copy(s_hbm.at[pl.ds(o, chunk)], c_v, semC)
        cpA.start()
        cpB.start()
        cpC.start()
        cpA.wait()
        cpB.wait()
        cpC.wait()

        def addb(i, carry):
            r = i >> 7
            c = i & 127
            sl = pl.ds(c * 16, 16)
            a_v[r, sl] = a_v[r, sl] + b_v[r, sl] + c_v[r, sl]
            return carry

        lax.fori_loop(0, chunk * (D // 16), addb, 0)
        pltpu.sync_copy(a_v, out_hbm.at[pl.ds(o, chunk)])


def _run_combine(h, shared, pos0, pos1):
    mesh = plsc.VectorSubcoreMesh(core_axis_name="c", subcore_axis_name="s")
    f = pl.kernel(
        _combine_body,
        out_type=jax.ShapeDtypeStruct((T, D), jnp.float32),
        mesh=mesh,
        scratch_types=[pltpu.VMEM((16,), jnp.int32),
                       pltpu.VMEM((16,), jnp.int32),
                       pltpu.VMEM((16, D), jnp.float32),
                       pltpu.VMEM((16, D), jnp.float32),
                       pltpu.VMEM((16, D), jnp.float32),
                       pltpu.SemaphoreType.DMA,
                       pltpu.SemaphoreType.DMA,
                       pltpu.SemaphoreType.DMA],
    )
    return f(h, shared, pos0, pos1)


def _shared_body(x_ref, sg_ref, su_ref, sd_ref, out_ref):
    xb = x_ref[...]
    g = lax.dot_general(xb, sg_ref[...], (((1,), (1,)), ((), ())),
                        preferred_element_type=jnp.float32,
                        precision=lax.Precision.DEFAULT)
    u = lax.dot_general(xb, su_ref[...], (((1,), (1,)), ((), ())),
                        preferred_element_type=jnp.float32,
                        precision=lax.Precision.DEFAULT)
    h = g * jax.nn.sigmoid(g) * u
    o = lax.dot_general(h, sd_ref[...], (((1,), (1,)), ((), ())),
                        preferred_element_type=jnp.float32,
                        precision=lax.Precision.DEFAULT)
    out_ref[...] = o


def _run_shared(x2, Sg, Su, Sd):
    return pl.pallas_call(
        _shared_body,
        grid=(T // TILE,),
        in_specs=[pl.BlockSpec((TILE, D), lambda i: (i, 0)),
                  pl.BlockSpec((FF, D), lambda i: (0, 0)),
                  pl.BlockSpec((FF, D), lambda i: (0, 0)),
                  pl.BlockSpec((D, FF), lambda i: (0, 0))],
        out_specs=pl.BlockSpec((TILE, D), lambda i: (i, 0)),
        out_shape=jax.ShapeDtypeStruct((T, D), jnp.float32),
    )(x2, Sg, Su, Sd)


def kernel(hidden_states, router_w, bias, Wg, Wu, Wd, Sg, Su, Sd):
    orig_shape = hidden_states.shape
    x2 = hidden_states.reshape(T, D)
    i1, i2, w1, w2 = _run_router(x2, router_w, bias.reshape(1, E))
    shared = _run_shared(x2, Sg, Su, Sd)
    pos0, pos1, te_col = _run_plan(i1, i2)
    row_token, row_gate = _run_invert(pos0, pos1, w1, w2)
    xs = _run_gather(x2, row_token.reshape(PROWS))
    h = _run_ffn(te_col.reshape(NTILES), xs, row_gate, Wg, Wu, Wd)
    out = _run_combine(h, shared, pos0.reshape(T), pos1.reshape(T))
    return out.reshape(orig_shape)


# speedup vs baseline: 1.5509x; 1.0881x over previous
import functools

import jax
import jax.numpy as jnp
from jax import lax
from jax.experimental import pallas as pl
from jax.experimental.pallas import tpu as pltpu
from jax.experimental.pallas import tpu_sc as plsc

E = 16
TOPK = 2
NGROUP = 4
GSIZE = E // NGROUP
TOPK_GROUP = 2
D = 2048
FF = 1024
SCALE = 2.5
T = 2048
P = T * TOPK
TILE = 128
PROWS = P + E * TILE
NTILES = PROWS // TILE
TB = 256
NSC = 32

_NEG = -1e30


def _router_body(x_ref, rw_ref, bias_ref, i1_ref, i2_ref, w1_ref, w2_ref):
    xb = x_ref[...]
    rw = rw_ref[...]
    logits = lax.dot_general(xb, rw, (((1,), (1,)), ((), ())),
                             preferred_element_type=jnp.float32)
    scores = jax.nn.sigmoid(logits)
    s4c = scores + bias_ref[...]
    lanes = lax.broadcasted_iota(jnp.int32, (TB, E), 1)

    def top1(vals):
        m = jnp.max(vals, axis=1, keepdims=True)
        i = jnp.min(jnp.where(vals == m, lanes, E), axis=1, keepdims=True)
        return m, i

    gs = jnp.where(lanes < NGROUP, 0.0, _NEG)
    for g in range(NGROUP):
        vals = jnp.where(lanes // GSIZE == g, s4c, _NEG)
        m1, i1 = top1(vals)
        m2, _ = top1(jnp.where(lanes == i1, _NEG, vals))
        gs = jnp.where(lanes == g, m1 + m2, gs)
    _, gi1 = top1(gs)
    _, gi2 = top1(jnp.where(lanes == gi1, _NEG, gs))
    sel = (lanes // GSIZE == gi1) | (lanes // GSIZE == gi2)
    masked = jnp.where(sel, s4c, 0.0)
    m1, i1 = top1(masked)
    _, i2 = top1(jnp.where(lanes == i1, _NEG, masked))
    w1 = jnp.sum(jnp.where(lanes == i1, scores, 0.0), axis=1, keepdims=True)
    w2 = jnp.sum(jnp.where(lanes == i2, scores, 0.0), axis=1, keepdims=True)
    norm = SCALE / (w1 + w2 + 1e-20)
    i1_ref[...] = i1
    i2_ref[...] = i2
    w1_ref[...] = w1 * norm
    w2_ref[...] = w2 * norm


def _run_router(x2, router_w, bias2):
    outs = jax.ShapeDtypeStruct((T, 1), jnp.int32), jax.ShapeDtypeStruct((T, 1), jnp.int32), \
        jax.ShapeDtypeStruct((T, 1), jnp.float32), jax.ShapeDtypeStruct((T, 1), jnp.float32)
    col = pl.BlockSpec((TB, 1), lambda i: (i, 0))
    return pl.pallas_call(
        _router_body,
        grid=(T // TB,),
        in_specs=[pl.BlockSpec((TB, D), lambda i: (i, 0)),
                  pl.BlockSpec((E, D), lambda i: (0, 0)),
                  pl.BlockSpec((1, E), lambda i: (0, 0))],
        out_specs=(col, col, col, col),
        out_shape=outs,
    )(x2, router_w, bias2)


def _plan_body(i1_ref, i2_ref, pos0_ref, pos1_ref, te_ref):
    lanes = lax.broadcasted_iota(jnp.int32, (T, E), 1)
    ohA = (i1_ref[...] == lanes).astype(jnp.int32)
    ohB = (i2_ref[...] == lanes).astype(jnp.int32)
    S = ohA + ohB
    counts = jnp.sum(S, axis=0, keepdims=True)
    padded = ((counts + (TILE - 1)) >> 7) << 7
    r16 = lax.broadcasted_iota(jnp.int32, (E, E), 0)
    c16 = lax.broadcasted_iota(jnp.int32, (E, E), 1)
    lt = (r16 <= c16).astype(jnp.float32)
    seg_incl = lax.dot_general(padded.astype(jnp.float32), lt,
                               (((1,), (0,)), ((), ())),
                               preferred_element_type=jnp.float32,
                               precision=lax.Precision.HIGHEST).astype(jnp.int32)
    seg_start = seg_incl - padded
    csum = S
    sh = 1
    while sh < T:
        shifted = jnp.concatenate(
            [jnp.zeros((sh, E), jnp.int32), csum[: T - sh, :]], axis=0)
        csum = csum + shifted
        sh *= 2
    excl = csum - S
    rank0 = jnp.sum(ohA * excl, axis=1, keepdims=True)
    rank1 = jnp.sum(ohB * excl, axis=1, keepdims=True)
    seg0 = jnp.sum(ohA * seg_start, axis=1, keepdims=True)
    seg1 = jnp.sum(ohB * seg_start, axis=1, keepdims=True)
    pos0_ref[...] = seg0 + rank0
    pos1_ref[...] = seg1 + rank1
    ts = lax.broadcasted_iota(jnp.int32, (NTILES, E), 0) * TILE
    te = jnp.sum((ts >= seg_incl).astype(jnp.int32), axis=1, keepdims=True)
    te_ref[...] = jnp.minimum(te, E - 1)


def _run_plan(i1, i2):
    col = pl.BlockSpec((T, 1), lambda: (0, 0))
    return pl.pallas_call(
        _plan_body,
        in_specs=[col, col],
        out_specs=(col, col, pl.BlockSpec((NTILES, 1), lambda: (0, 0))),
        out_shape=(jax.ShapeDtypeStruct((T, 1), jnp.int32),
                   jax.ShapeDtypeStruct((T, 1), jnp.int32),
                   jax.ShapeDtypeStruct((NTILES, 1), jnp.int32)),
    )(i1, i2)


def _invert_body(p0r_ref, p1r_ref, g1r_ref, g2r_ref, rt_ref, rg_ref):
    i = pl.program_id(0)
    rowid = i * TILE + lax.broadcasted_iota(jnp.int32, (TILE, T), 0)
    oh0 = p0r_ref[...] == rowid
    oh1 = p1r_ref[...] == rowid
    ohe = oh0 | oh1
    lanesT = lax.broadcasted_iota(jnp.int32, (TILE, T), 1)
    tok = jnp.sum(jnp.where(ohe, lanesT, 0), axis=1, keepdims=True)
    matched = jnp.sum(ohe.astype(jnp.int32), axis=1, keepdims=True)
    rowcol = i * TILE + lax.broadcasted_iota(jnp.int32, (TILE, 1), 0)
    rt_ref[...] = jnp.where(matched == 0, rowcol & (T - 1), tok)
    gate = jnp.sum(jnp.where(oh0, g1r_ref[...], 0.0)
                   + jnp.where(oh1, g2r_ref[...], 0.0), axis=1, keepdims=True)
    rg_ref[...] = gate


def _run_invert(pos0, pos1, w1, w2):
    row = pl.BlockSpec((1, T), lambda i: (0, 0))
    blk = pl.BlockSpec((TILE, 1), lambda i: (i, 0))
    return pl.pallas_call(
        _invert_body,
        grid=(NTILES,),
        in_specs=[row, row, row, row],
        out_specs=(blk, blk),
        out_shape=(jax.ShapeDtypeStruct((PROWS, 1), jnp.int32),
                   jax.ShapeDtypeStruct((PROWS, 1), jnp.float32)),
    )(pos0.reshape(1, T), pos1.reshape(1, T),
      w1.reshape(1, T), w2.reshape(1, T))


def _gather_body(x_hbm, rt_hbm, xs_hbm, raw_v, idx_v, rows_v, sems):
    wid = lax.axis_index("s") * 2 + lax.axis_index("c")
    per = PROWS // NSC
    chunk = 16
    nch = per // chunk
    base = wid * per
    pltpu.sync_copy(rt_hbm.at[pl.ds(base, per)], raw_v)
    for c in range(nch):
        idx_v[pl.ds(c * chunk, chunk)] = raw_v[pl.ds(c * chunk, chunk)] & (T - 1)

    def fire(ci, b):
        cp = pltpu.make_async_copy(
            x_hbm.at[idx_v.at[pl.ds(ci * chunk, chunk)]], rows_v.at[b],
            sems.at[b])
        cp.start()
        return cp

    cps = {0: fire(0, 0)}
    for ci in range(nch):
        b = ci % 2
        cps[ci].wait()
        if ci + 1 < nch:
            cps[ci + 1] = fire(ci + 1, 1 - b)
        pltpu.sync_copy(rows_v.at[b], xs_hbm.at[pl.ds(base + ci * chunk, chunk)])


def _run_gather(x2, row_token):
    mesh = plsc.VectorSubcoreMesh(core_axis_name="c", subcore_axis_name="s")
    per = PROWS // NSC
    f = pl.kernel(
        _gather_body,
        out_type=jax.ShapeDtypeStruct((PROWS, D), jnp.float32),
        mesh=mesh,
        scratch_types=[pltpu.VMEM((per,), jnp.int32),
                       pltpu.VMEM((per,), jnp.int32),
                       pltpu.VMEM((2, 16, D), jnp.float32),
                       pltpu.SemaphoreType.DMA((2,))],
    )
    return f(x2, row_token)


def _ffn_body(te_ref, xs_ref, gate_ref, wg_ref, wu_ref, wd_ref, out_ref):
    xb = xs_ref[...]
    g = lax.dot_general(xb, wg_ref[0], (((1,), (1,)), ((), ())),
                        preferred_element_type=jnp.float32,
                        precision=lax.Precision.DEFAULT)
    u = lax.dot_general(xb, wu_ref[0], (((1,), (1,)), ((), ())),
                        preferred_element_type=jnp.float32,
                        precision=lax.Precision.DEFAULT)
    h = g * jax.nn.sigmoid(g) * u
    o = lax.dot_general(h, wd_ref[0], (((1,), (1,)), ((), ())),
                        preferred_element_type=jnp.float32,
                        precision=lax.Precision.DEFAULT)
    out_ref[...] = o * gate_ref[...]


def _run_ffn(te, xs, row_gate_col, Wg, Wu, Wd):
    spec = pltpu.PrefetchScalarGridSpec(
        num_scalar_prefetch=1,
        grid=(NTILES,),
        in_specs=[
            pl.BlockSpec((TILE, D), lambda i, te: (i, 0)),
            pl.BlockSpec((TILE, 1), lambda i, te: (i, 0)),
            pl.BlockSpec((1, FF, D), lambda i, te: (te[i], 0, 0)),
            pl.BlockSpec((1, FF, D), lambda i, te: (te[i], 0, 0)),
            pl.BlockSpec((1, D, FF), lambda i, te: (te[i], 0, 0)),
        ],
        out_specs=pl.BlockSpec((TILE, D), lambda i, te: (i, 0)),
    )
    return pl.pallas_call(
        _ffn_body,
        grid_spec=spec,
        out_shape=jax.ShapeDtypeStruct((PROWS, D), jnp.float32),
    )(te, xs, row_gate_col, Wg, Wu, Wd)


def _combine_body(h_hbm, s_hbm, p0_hbm, p1_hbm, out_hbm, p0_v, p1_v, a_v,
                  b_v, c_v, semA, semB, semC):
    wid = lax.axis_index("s") * 2 + lax.axis_index("c")
    per = T // NSC
    chunk = 8
    nch = per // chunk
    base = wid * per
    pltpu.sync_copy(p0_hbm.at[pl.ds(base, per)], p0_v)
    pltpu.sync_copy(p1_hbm.at[pl.ds(base, per)], p1_v)

    def fire(ci, b):
        sl = pl.ds(ci * chunk, chunk)
        cpA = pltpu.make_async_copy(h_hbm.at[p0_v.at[sl]], a_v.at[b],
                                    semA.at[b])
        cpB = pltpu.make_async_copy(h_hbm.at[p1_v.at[sl]], b_v.at[b],
                                    semB.at[b])
        cpC = pltpu.make_async_copy(s_hbm.at[pl.ds(base + ci * chunk, chunk)],
                                    c_v.at[b], semC.at[b])
        cpA.start()
        cpB.start()
        cpC.start()
        return (cpA, cpB, cpC)

    cps = {0: fire(0, 0)}
    for ci in range(nch):
        b = ci % 2
        for cp in cps[ci]:
            cp.wait()
        if ci + 1 < nch:
            cps[ci + 1] = fire(ci + 1, 1 - b)

        @plsc.parallel_loop(0, chunk * (D // 16), unroll=8)
        def _add(i):
            r = i >> 7
            c = i & 127
            sl = pl.ds(c * 16, 16)
            a_v[b, r, sl] = a_v[b, r, sl] + b_v[b, r, sl] + c_v[b, r, sl]

        pltpu.sync_copy(a_v.at[b],
                        out_hbm.at[pl.ds(base + ci * chunk, chunk)])


def _run_combine(h, shared, pos0, pos1):
    mesh = plsc.VectorSubcoreMesh(core_axis_name="c", subcore_axis_name="s")
    per = T // NSC
    f = pl.kernel(
        _combine_body,
        out_type=jax.ShapeDtypeStruct((T, D), jnp.float32),
        mesh=mesh,
        scratch_types=[pltpu.VMEM((per,), jnp.int32),
                       pltpu.VMEM((per,), jnp.int32),
                       pltpu.VMEM((2, 8, D), jnp.float32),
                       pltpu.VMEM((2, 8, D), jnp.float32),
                       pltpu.VMEM((2, 8, D), jnp.float32),
                       pltpu.SemaphoreType.DMA((2,)),
                       pltpu.SemaphoreType.DMA((2,)),
                       pltpu.SemaphoreType.DMA((2,))],
    )
    return f(h, shared, pos0, pos1)


def _shared_body(x_ref, sg_ref, su_ref, sd_ref, out_ref):
    xb = x_ref[...]
    g = lax.dot_general(xb, sg_ref[...], (((1,), (1,)), ((), ())),
                        preferred_element_type=jnp.float32,
                        precision=lax.Precision.DEFAULT)
    u = lax.dot_general(xb, su_ref[...], (((1,), (1,)), ((), ())),
                        preferred_element_type=jnp.float32,
                        precision=lax.Precision.DEFAULT)
    h = g * jax.nn.sigmoid(g) * u
    o = lax.dot_general(h, sd_ref[...], (((1,), (1,)), ((), ())),
                        preferred_element_type=jnp.float32,
                        precision=lax.Precision.DEFAULT)
    out_ref[...] = o


def _run_shared(x2, Sg, Su, Sd):
    return pl.pallas_call(
        _shared_body,
        grid=(T // TILE,),
        in_specs=[pl.BlockSpec((TILE, D), lambda i: (i, 0)),
                  pl.BlockSpec((FF, D), lambda i: (0, 0)),
                  pl.BlockSpec((FF, D), lambda i: (0, 0)),
                  pl.BlockSpec((D, FF), lambda i: (0, 0))],
        out_specs=pl.BlockSpec((TILE, D), lambda i: (i, 0)),
        out_shape=jax.ShapeDtypeStruct((T, D), jnp.float32),
    )(x2, Sg, Su, Sd)


def kernel(hidden_states, router_w, bias, Wg, Wu, Wd, Sg, Su, Sd):
    orig_shape = hidden_states.shape
    x2 = hidden_states.reshape(T, D)
    i1, i2, w1, w2 = _run_router(x2, router_w, bias.reshape(1, E))
    shared = _run_shared(x2, Sg, Su, Sd)
    pos0, pos1, te_col = _run_plan(i1, i2)
    row_token, row_gate = _run_invert(pos0, pos1, w1, w2)
    xs = _run_gather(x2, row_token.reshape(PROWS))
    h = _run_ffn(te_col.reshape(NTILES), xs, row_gate, Wg, Wu, Wd)
    out = _run_combine(h, shared, pos0.reshape(T), pos1.reshape(T))
    return out.reshape(orig_shape)
